# baseline (speedup 1.0000x reference)
import functools

import jax
import jax.numpy as jnp
from jax import lax
from jax.experimental import pallas as pl
from jax.experimental.pallas import tpu as pltpu
from jax.experimental.pallas import tpu_sc as plsc

_N = 16384
_C = 1000
_D = 128
_K = _N // 2


_BN = 1024
_CP = 1024


def _loss_body(s1_ref, s2_ref, lab_ref, loss_ref, labrm_ref):
    lab = lab_ref[...]
    a = jnp.sum(lab * jnp.log(s2_ref[...]), axis=0)
    b = jnp.sum(lab * jnp.log(s1_ref[...]), axis=0)
    loss_ref[...] = jnp.abs(-a + b)
    labrm_ref[...] = jnp.concatenate(
        [lab.T, jnp.zeros((_BN, _CP - _C), jnp.float32)], axis=1)


def _loss_fn(score1, score2, label):
    return pl.pallas_call(
        _loss_body,
        grid=(_N // _BN,),
        in_specs=[pl.BlockSpec((_C, _BN), lambda i: (0, i))] * 3,
        out_specs=[
            pl.BlockSpec((_BN,), lambda i: (i,)),
            pl.BlockSpec((_BN, _CP), lambda i: (i, 0)),
        ],
        out_shape=[
            jax.ShapeDtypeStruct((_N,), jnp.float32),
            jax.ShapeDtypeStruct((_N, _CP), jnp.float32),
        ],
    )(score1.T, score2.T, label.T)


_NT = 16
_EPT = _N // _NT
_NB = 2048
_BAT = 64


def _radix_body(key_hbm, idx_hbm, cur_k, cur_v, counters, allhist,
                posb, kb, vb, obuf, hists_sh, s1k, s1v, s2k, s2v, sem):
    cid = lax.axis_index("c")
    sid = lax.axis_index("s")

    @pl.when(cid == 0)
    def _radix():
        base = sid * _EPT
        zero16 = jnp.zeros((16,), jnp.int32)
        iota16 = lax.iota(jnp.int32, 16)
        calr, _ = plsc.scan_count(zero16)
        adj = calr[0]

        pltpu.sync_copy(key_hbm.at[pl.ds(base, _EPT)], cur_k)

        def vinit(t, carry):
            cur_v[pl.ds(t * 16, 16)] = base + t * 16 + iota16
            return carry

        lax.fori_loop(0, _EPT // 16, vinit, 0)

        def run_pass(digit_fn, dst_k, dst_v, last):
            def zstep(t, carry):
                counters[pl.ds(t * 16, 16)] = zero16
                return carry

            lax.fori_loop(0, _NB // 16, zstep, 0)

            def hstep(t, carry):
                d = digit_fn(cur_k[pl.ds(t * 16, 16)])
                dupr, islast = plsc.scan_count(d)
                cnt = plsc.load_gather(counters, [d])
                plsc.store_scatter(counters, [d], cnt + (dupr - adj) + 1,
                                   mask=islast)
                return carry

            lax.fori_loop(0, _EPT // 16, hstep, 0)

            pltpu.sync_copy(counters, hists_sh.at[sid])
            plsc.subcore_barrier()
            pltpu.sync_copy(hists_sh, allhist)

            def ostep(c, carry):
                sl = pl.ds(c * 16, 16)
                tot = zero16
                below = zero16
                for t in range(_NT):
                    row = allhist[t, sl]
                    tot = tot + row
                    below = below + jnp.where(t < sid, row, 0)
                incl = plsc.cumsum(tot)
                counters[sl] = (incl - tot) + below + carry
                return carry + jnp.sum(tot)

            lax.fori_loop(0, _NB // 16, ostep, 0)

            def bstep(b, carry):
                for m in range(_BAT // 16):
                    sl = pl.ds(b * _BAT + m * 16, 16)
                    osl = pl.ds(m * 16, 16)
                    ke = cur_k[sl]
                    ve = cur_v[sl]
                    d = digit_fn(ke)
                    dupr, islast = plsc.scan_count(d)
                    cnt = plsc.load_gather(counters, [d])
                    pos = cnt + (dupr - adj)
                    plsc.store_scatter(counters, [d], pos + 1, mask=islast)
                    posb[0, osl] = pos
                    if not last:
                        kb[0, osl] = ke
                    vb[0, osl] = ve
                if not last:
                    pltpu.sync_copy(kb.at[0], dst_k.at[posb.at[0]])
                pltpu.sync_copy(vb.at[0], dst_v.at[posb.at[0]])
                return carry

            lax.fori_loop(0, _EPT // _BAT, bstep, 0)
            plsc.subcore_barrier()

        run_pass(lambda k: k & (_NB - 1), s1k, s1v, False)
        pltpu.sync_copy(s1k.at[pl.ds(base, _EPT)], cur_k)
        pltpu.sync_copy(s1v.at[pl.ds(base, _EPT)], cur_v)
        plsc.subcore_barrier()
        run_pass(lambda k: (k >> 11) & (_NB - 1), s2k, s2v, False)
        pltpu.sync_copy(s2k.at[pl.ds(base, _EPT)], cur_k)
        pltpu.sync_copy(s2v.at[pl.ds(base, _EPT)], cur_v)
        plsc.subcore_barrier()
        run_pass(lambda k: k >> 22, s1k, s1v, True)

        pltpu.sync_copy(s1v.at[pl.ds(sid * (_K // _NT), _K // _NT)], obuf)
        pltpu.sync_copy(obuf, idx_hbm.at[pl.ds(sid * (_K // _NT), _K // _NT)])


@functools.cache
def _radix_sort():
    return functools.partial(
        pl.kernel,
        mesh=plsc.VectorSubcoreMesh(core_axis_name="c", subcore_axis_name="s"),
        out_type=jax.ShapeDtypeStruct((_K,), jnp.int32),
        scratch_types=[
            pltpu.VMEM((_EPT,), jnp.int32),
            pltpu.VMEM((_EPT,), jnp.int32),
            pltpu.VMEM((_NB,), jnp.int32),
            pltpu.VMEM((_NT, _NB), jnp.int32),
            pltpu.VMEM((1, _BAT), jnp.int32),
            pltpu.VMEM((1, _BAT), jnp.int32),
            pltpu.VMEM((1, _BAT), jnp.int32),
            pltpu.VMEM((_K // _NT,), jnp.int32),
            pltpu.VMEM_SHARED((_NT, _NB), jnp.int32),
            pltpu.VMEM_SHARED((_N,), jnp.int32),
            pltpu.VMEM_SHARED((_N,), jnp.int32),
            pltpu.VMEM_SHARED((_N,), jnp.int32),
            pltpu.VMEM_SHARED((_N,), jnp.int32),
            pltpu.SemaphoreType.DMA,
        ],
        compiler_params=pltpu.CompilerParams(needs_layout_passes=False),
    )(_radix_body)


_NW = 32
_BPW = _K // _NW
_CH = 32
_CH1 = 128


def _gather_body(idx_hbm, in2_hbm, lab_hbm, out1_hbm, out2_hbm,
                 idx_v, rows1_v, rows2_v, sem1, sem2a, sem2b):
    wid = lax.axis_index("s") * 2 + lax.axis_index("c")
    base = wid * _BPW
    pltpu.sync_copy(idx_hbm.at[pl.ds(base, _BPW)], idx_v)

    in2_descs = [
        pltpu.async_copy(in2_hbm.at[idx_v.at[pl.ds(c * _CH1, _CH1)]],
                         rows1_v.at[pl.ds(c * _CH1, _CH1)], sem1)
        for c in range(_BPW // _CH1)
    ]

    sems = (sem2a, sem2b)
    prev = None
    for c in range(_BPW // _CH):
        d = pltpu.async_copy(lab_hbm.at[idx_v.at[pl.ds(c * _CH, _CH)]],
                             rows2_v.at[c % 2], sems[c % 2])
        if prev is not None:
            pc, pd = prev
            pd.wait()
            pltpu.sync_copy(rows2_v.at[pc % 2],
                            out2_hbm.at[pl.ds(base + pc * _CH, _CH)])
        prev = (c, d)
    pc, pd = prev
    pd.wait()
    pltpu.sync_copy(rows2_v.at[pc % 2], out2_hbm.at[pl.ds(base + pc * _CH, _CH)])

    for d in in2_descs:
        d.wait()
    pltpu.sync_copy(rows1_v, out1_hbm.at[pl.ds(base, _BPW)])


@functools.cache
def _sel_gather():
    return functools.partial(
        pl.kernel,
        mesh=plsc.VectorSubcoreMesh(core_axis_name="c", subcore_axis_name="s"),
        out_type=(
            jax.ShapeDtypeStruct((_K, _D), jnp.float32),
            jax.ShapeDtypeStruct((_K, _CP), jnp.float32),
        ),
        scratch_types=[
            pltpu.VMEM((_BPW,), jnp.int32),
            pltpu.VMEM((_BPW, _D), jnp.float32),
            pltpu.VMEM((2, _CH, _CP), jnp.float32),
            pltpu.SemaphoreType.DMA,
            pltpu.SemaphoreType.DMA,
            pltpu.SemaphoreType.DMA,
        ],
        compiler_params=pltpu.CompilerParams(needs_layout_passes=False),
    )(_gather_body)


_BT = 512


def _xpose_body(in_ref, out_ref):
    out_ref[...] = in_ref[...].T


def _xpose_fn(labelss_p):
    return pl.pallas_call(
        _xpose_body,
        grid=(_K // _BT,),
        in_specs=[pl.BlockSpec((_BT, _CP), lambda i: (i, 0))],
        out_specs=pl.BlockSpec((_CP, _BT), lambda i: (0, i)),
        out_shape=jax.ShapeDtypeStruct((_CP, _K), jnp.float32),
    )(labelss_p)


def kernel(input1, input2, score1, score2, label):
    del input1
    loss, labrm = _loss_fn(score1, score2, label)
    key = lax.bitcast_convert_type(loss, jnp.int32)
    idx = _radix_sort()(key)
    inputss, labelss_p = _sel_gather()(idx, input2, labrm)
    labelss = _xpose_fn(labelss_p)[:_C].T
    return (inputss, labelss)


# speedup vs baseline: 1.1174x; 1.1174x over previous
import functools

import jax
import jax.numpy as jnp
from jax import lax
from jax.experimental import pallas as pl
from jax.experimental.pallas import tpu as pltpu
from jax.experimental.pallas import tpu_sc as plsc

_N = 16384
_C = 1000
_D = 128
_K = _N // 2


_BN = 1024
_CP = 1024


def _loss_body(s1_ref, s2_ref, lab_ref, loss_ref, labrm_ref):
    lab = lab_ref[...]
    a = jnp.sum(lab * jnp.log(s2_ref[...]), axis=0)
    b = jnp.sum(lab * jnp.log(s1_ref[...]), axis=0)
    loss_ref[...] = jnp.abs(-a + b)
    labrm_ref[...] = jnp.concatenate(
        [lab.T, jnp.zeros((_BN, _CP - _C), jnp.float32)], axis=1)


def _loss_fn(score1, score2, label):
    return pl.pallas_call(
        _loss_body,
        grid=(_N // _BN,),
        in_specs=[pl.BlockSpec((_C, _BN), lambda i: (0, i))] * 3,
        out_specs=[
            pl.BlockSpec((_BN,), lambda i: (i,)),
            pl.BlockSpec((_BN, _CP), lambda i: (i, 0)),
        ],
        out_shape=[
            jax.ShapeDtypeStruct((_N,), jnp.float32),
            jax.ShapeDtypeStruct((_N, _CP), jnp.float32),
        ],
    )(score1.T, score2.T, label.T)


_NT = 16
_EPT = _N // _NT
_NB = 2048
_BAT = 64


def _radix_body(key_hbm, idx_hbm, cur_k, cur_v, counters, allhist,
                posb, kb, vb, obuf, hists_sh, s1k, s1v, s2k, s2v, sem):
    cid = lax.axis_index("c")
    sid = lax.axis_index("s")

    @pl.when(cid == 0)
    def _radix():
        base = sid * _EPT
        zero16 = jnp.zeros((16,), jnp.int32)
        iota16 = lax.iota(jnp.int32, 16)
        calr, _ = plsc.scan_count(zero16)
        adj = calr[0]

        pltpu.sync_copy(key_hbm.at[pl.ds(base, _EPT)], cur_k)

        def vinit(t, carry):
            cur_v[pl.ds(t * 16, 16)] = base + t * 16 + iota16
            return carry

        lax.fori_loop(0, _EPT // 16, vinit, 0)

        def run_pass(digit_fn, dst_k, dst_v, last):
            def zstep(t, carry):
                counters[pl.ds(t * 16, 16)] = zero16
                return carry

            lax.fori_loop(0, _NB // 16, zstep, 0)

            def hstep(t, carry):
                d = digit_fn(cur_k[pl.ds(t * 16, 16)])
                dupr, islast = plsc.scan_count(d)
                cnt = plsc.load_gather(counters, [d])
                plsc.store_scatter(counters, [d], cnt + (dupr - adj) + 1,
                                   mask=islast)
                return carry

            lax.fori_loop(0, _EPT // 16, hstep, 0)

            pltpu.sync_copy(counters, hists_sh.at[sid])
            plsc.subcore_barrier()
            pltpu.sync_copy(hists_sh, allhist)

            def ostep(c, carry):
                sl = pl.ds(c * 16, 16)
                tot = zero16
                below = zero16
                for t in range(_NT):
                    row = allhist[t, sl]
                    tot = tot + row
                    below = below + jnp.where(t < sid, row, 0)
                incl = plsc.cumsum(tot)
                counters[sl] = (incl - tot) + below + carry
                return carry + jnp.sum(tot)

            lax.fori_loop(0, _NB // 16, ostep, 0)

            def bstep(b, carry):
                for m in range(_BAT // 16):
                    sl = pl.ds(b * _BAT + m * 16, 16)
                    osl = pl.ds(m * 16, 16)
                    ke = cur_k[sl]
                    ve = cur_v[sl]
                    d = digit_fn(ke)
                    dupr, islast = plsc.scan_count(d)
                    cnt = plsc.load_gather(counters, [d])
                    pos = cnt + (dupr - adj)
                    plsc.store_scatter(counters, [d], pos + 1, mask=islast)
                    posb[0, osl] = pos
                    if not last:
                        kb[0, osl] = ke
                    vb[0, osl] = ve
                if not last:
                    pltpu.sync_copy(kb.at[0], dst_k.at[posb.at[0]])
                pltpu.sync_copy(vb.at[0], dst_v.at[posb.at[0]])
                return carry

            lax.fori_loop(0, _EPT // _BAT, bstep, 0)
            plsc.subcore_barrier()

        run_pass(lambda k: k & (_NB - 1), s1k, s1v, False)
        pltpu.sync_copy(s1k.at[pl.ds(base, _EPT)], cur_k)
        pltpu.sync_copy(s1v.at[pl.ds(base, _EPT)], cur_v)
        plsc.subcore_barrier()
        run_pass(lambda k: (k >> 11) & (_NB - 1), s2k, s2v, False)
        pltpu.sync_copy(s2k.at[pl.ds(base, _EPT)], cur_k)
        pltpu.sync_copy(s2v.at[pl.ds(base, _EPT)], cur_v)
        plsc.subcore_barrier()
        run_pass(lambda k: k >> 22, s1k, s1v, True)

        pltpu.sync_copy(s1v.at[pl.ds(sid * (_K // _NT), _K // _NT)], obuf)
        pltpu.sync_copy(obuf, idx_hbm.at[pl.ds(sid * (_K // _NT), _K // _NT)])


@functools.cache
def _radix_sort():
    return functools.partial(
        pl.kernel,
        mesh=plsc.VectorSubcoreMesh(core_axis_name="c", subcore_axis_name="s"),
        out_type=jax.ShapeDtypeStruct((_K,), jnp.int32),
        scratch_types=[
            pltpu.VMEM((_EPT,), jnp.int32),
            pltpu.VMEM((_EPT,), jnp.int32),
            pltpu.VMEM((_NB,), jnp.int32),
            pltpu.VMEM((_NT, _NB), jnp.int32),
            pltpu.VMEM((1, _BAT), jnp.int32),
            pltpu.VMEM((1, _BAT), jnp.int32),
            pltpu.VMEM((1, _BAT), jnp.int32),
            pltpu.VMEM((_K // _NT,), jnp.int32),
            pltpu.VMEM_SHARED((_NT, _NB), jnp.int32),
            pltpu.VMEM_SHARED((_N,), jnp.int32),
            pltpu.VMEM_SHARED((_N,), jnp.int32),
            pltpu.VMEM_SHARED((_N,), jnp.int32),
            pltpu.VMEM_SHARED((_N,), jnp.int32),
            pltpu.SemaphoreType.DMA,
        ],
        compiler_params=pltpu.CompilerParams(needs_layout_passes=False),
    )(_radix_body)


_NW = 32
_BPW = _K // _NW
_CH = 32
_CH1 = 128


def _gather_body(idx_hbm, in2_hbm, lab_hbm, out1_hbm, out2_hbm,
                 idx_v, rows1_v, rows2_v, sem1, sem2a, sem2b):
    wid = lax.axis_index("s") * 2 + lax.axis_index("c")
    base = wid * _BPW
    pltpu.sync_copy(idx_hbm.at[pl.ds(base, _BPW)], idx_v)

    in2_descs = [
        pltpu.async_copy(in2_hbm.at[idx_v.at[pl.ds(c * _CH1, _CH1)]],
                         rows1_v.at[pl.ds(c * _CH1, _CH1)], sem1)
        for c in range(_BPW // _CH1)
    ]

    sems = (sem2a, sem2b)
    prev = None
    for c in range(_BPW // _CH):
        d = pltpu.async_copy(lab_hbm.at[idx_v.at[pl.ds(c * _CH, _CH)]],
                             rows2_v.at[c % 2], sems[c % 2])
        if prev is not None:
            pc, pd = prev
            pd.wait()
            pltpu.sync_copy(rows2_v.at[pc % 2],
                            out2_hbm.at[pl.ds(base + pc * _CH, _CH)])
        prev = (c, d)
    pc, pd = prev
    pd.wait()
    pltpu.sync_copy(rows2_v.at[pc % 2], out2_hbm.at[pl.ds(base + pc * _CH, _CH)])

    for d in in2_descs:
        d.wait()
    pltpu.sync_copy(rows1_v, out1_hbm.at[pl.ds(base, _BPW)])


@functools.cache
def _sel_gather():
    return functools.partial(
        pl.kernel,
        mesh=plsc.VectorSubcoreMesh(core_axis_name="c", subcore_axis_name="s"),
        out_type=(
            jax.ShapeDtypeStruct((_K, _D), jnp.float32),
            jax.ShapeDtypeStruct((_K, _CP), jnp.float32),
        ),
        scratch_types=[
            pltpu.VMEM((_BPW,), jnp.int32),
            pltpu.VMEM((_BPW, _D), jnp.float32),
            pltpu.VMEM((2, _CH, _CP), jnp.float32),
            pltpu.SemaphoreType.DMA,
            pltpu.SemaphoreType.DMA,
            pltpu.SemaphoreType.DMA,
        ],
        compiler_params=pltpu.CompilerParams(needs_layout_passes=False),
    )(_gather_body)


def kernel(input1, input2, score1, score2, label):
    del input1
    loss, labrm = _loss_fn(score1, score2, label)
    key = lax.bitcast_convert_type(loss, jnp.int32)
    idx = _radix_sort()(key)
    inputss, labelss_p = _sel_gather()(idx, input2, labrm)
    return (inputss, labelss_p[:, :_C])


# speedup vs baseline: 1.1271x; 1.0086x over previous
import functools

import jax
import jax.numpy as jnp
from jax import lax
from jax.experimental import pallas as pl
from jax.experimental.pallas import tpu as pltpu
from jax.experimental.pallas import tpu_sc as plsc

_N = 16384
_C = 1000
_D = 128
_K = _N // 2


_BN = 1024
_CP = 1024


def _loss_body(s1_ref, s2_ref, lab_ref, loss_ref, labrm_ref):
    lab = lab_ref[...]
    a = jnp.sum(lab * jnp.log(s2_ref[...]), axis=0)
    b = jnp.sum(lab * jnp.log(s1_ref[...]), axis=0)
    loss_ref[...] = jnp.abs(-a + b)
    labrm_ref[...] = jnp.concatenate(
        [lab.T, jnp.zeros((_BN, _CP - _C), jnp.float32)], axis=1)


def _loss_fn(score1, score2, label):
    return pl.pallas_call(
        _loss_body,
        grid=(_N // _BN,),
        in_specs=[pl.BlockSpec((_C, _BN), lambda i: (0, i))] * 3,
        out_specs=[
            pl.BlockSpec((_BN,), lambda i: (i,)),
            pl.BlockSpec((_BN, _CP), lambda i: (i, 0)),
        ],
        out_shape=[
            jax.ShapeDtypeStruct((_N,), jnp.float32),
            jax.ShapeDtypeStruct((_N, _CP), jnp.float32),
        ],
    )(score1.T, score2.T, label.T)


_NT = 16
_EPT = _N // _NT
_NB = 2048
_BAT = 64


def _radix_body(key_hbm, idx_hbm, cur_k, cur_v, counters, allhist,
                posb, kb, vb, obuf, hists_sh, s1k, s1v, s2k, s2v, sem):
    cid = lax.axis_index("c")
    sid = lax.axis_index("s")

    @pl.when(cid == 0)
    def _radix():
        base = sid * _EPT
        zero16 = jnp.zeros((16,), jnp.int32)
        iota16 = lax.iota(jnp.int32, 16)
        calr, _ = plsc.scan_count(zero16)
        adj = calr[0]

        pltpu.sync_copy(key_hbm.at[pl.ds(base, _EPT)], cur_k)

        def vinit(t, carry):
            cur_v[pl.ds(t * 16, 16)] = base + t * 16 + iota16
            return carry

        lax.fori_loop(0, _EPT // 16, vinit, 0)

        def run_pass(digit_fn, dst_k, dst_v, last):
            def zstep(t, carry):
                counters[pl.ds(t * 16, 16)] = zero16
                return carry

            lax.fori_loop(0, _NB // 16, zstep, 0)

            def hstep(t, carry):
                d = digit_fn(cur_k[pl.ds(t * 16, 16)])
                dupr, islast = plsc.scan_count(d)
                cnt = plsc.load_gather(counters, [d])
                plsc.store_scatter(counters, [d], cnt + (dupr - adj) + 1,
                                   mask=islast)
                return carry

            lax.fori_loop(0, _EPT // 16, hstep, 0)

            pltpu.sync_copy(counters, hists_sh.at[sid])
            plsc.subcore_barrier()
            pltpu.sync_copy(hists_sh, allhist)

            def ostep(c, carry):
                sl = pl.ds(c * 16, 16)
                tot = zero16
                below = zero16
                for t in range(_NT):
                    row = allhist[t, sl]
                    tot = tot + row
                    below = below + jnp.where(t < sid, row, 0)
                incl = plsc.cumsum(tot)
                counters[sl] = (incl - tot) + below + carry
                return carry + jnp.sum(tot)

            lax.fori_loop(0, _NB // 16, ostep, 0)

            def bstep(b, carry):
                for m in range(_BAT // 16):
                    sl = pl.ds(b * _BAT + m * 16, 16)
                    osl = pl.ds(m * 16, 16)
                    ke = cur_k[sl]
                    ve = cur_v[sl]
                    d = digit_fn(ke)
                    dupr, islast = plsc.scan_count(d)
                    cnt = plsc.load_gather(counters, [d])
                    pos = cnt + (dupr - adj)
                    plsc.store_scatter(counters, [d], pos + 1, mask=islast)
                    posb[0, osl] = pos
                    if not last:
                        kb[0, osl] = ke
                    vb[0, osl] = ve
                descs = []
                if not last:
                    descs.append(
                        pltpu.async_copy(kb.at[0], dst_k.at[posb.at[0]], sem))
                descs.append(
                    pltpu.async_copy(vb.at[0], dst_v.at[posb.at[0]], sem))
                for d in descs:
                    d.wait()
                return carry

            lax.fori_loop(0, _EPT // _BAT, bstep, 0)
            plsc.subcore_barrier()

        run_pass(lambda k: k & (_NB - 1), s1k, s1v, False)
        pltpu.sync_copy(s1k.at[pl.ds(base, _EPT)], cur_k)
        pltpu.sync_copy(s1v.at[pl.ds(base, _EPT)], cur_v)
        plsc.subcore_barrier()
        run_pass(lambda k: (k >> 11) & (_NB - 1), s2k, s2v, False)
        pltpu.sync_copy(s2k.at[pl.ds(base, _EPT)], cur_k)
        pltpu.sync_copy(s2v.at[pl.ds(base, _EPT)], cur_v)
        plsc.subcore_barrier()
        run_pass(lambda k: k >> 22, s1k, s1v, True)

        pltpu.sync_copy(s1v.at[pl.ds(sid * (_K // _NT), _K // _NT)], obuf)
        pltpu.sync_copy(obuf, idx_hbm.at[pl.ds(sid * (_K // _NT), _K // _NT)])


@functools.cache
def _radix_sort():
    return functools.partial(
        pl.kernel,
        mesh=plsc.VectorSubcoreMesh(core_axis_name="c", subcore_axis_name="s"),
        out_type=jax.ShapeDtypeStruct((_K,), jnp.int32),
        scratch_types=[
            pltpu.VMEM((_EPT,), jnp.int32),
            pltpu.VMEM((_EPT,), jnp.int32),
            pltpu.VMEM((_NB,), jnp.int32),
            pltpu.VMEM((_NT, _NB), jnp.int32),
            pltpu.VMEM((1, _BAT), jnp.int32),
            pltpu.VMEM((1, _BAT), jnp.int32),
            pltpu.VMEM((1, _BAT), jnp.int32),
            pltpu.VMEM((_K // _NT,), jnp.int32),
            pltpu.VMEM_SHARED((_NT, _NB), jnp.int32),
            pltpu.VMEM_SHARED((_N,), jnp.int32),
            pltpu.VMEM_SHARED((_N,), jnp.int32),
            pltpu.VMEM_SHARED((_N,), jnp.int32),
            pltpu.VMEM_SHARED((_N,), jnp.int32),
            pltpu.SemaphoreType.DMA,
        ],
        compiler_params=pltpu.CompilerParams(needs_layout_passes=False),
    )(_radix_body)


_NW = 32
_BPW = _K // _NW
_CH = 32
_CH1 = 128


def _gather_body(idx_hbm, in2_hbm, lab_hbm, out1_hbm, out2_hbm,
                 idx_v, rows1_v, rows2_v, sem1, sem2a, sem2b):
    wid = lax.axis_index("s") * 2 + lax.axis_index("c")
    base = wid * _BPW
    pltpu.sync_copy(idx_hbm.at[pl.ds(base, _BPW)], idx_v)

    in2_descs = [
        pltpu.async_copy(in2_hbm.at[idx_v.at[pl.ds(c * _CH1, _CH1)]],
                         rows1_v.at[pl.ds(c * _CH1, _CH1)], sem1)
        for c in range(_BPW // _CH1)
    ]

    sems = (sem2a, sem2b)
    prev = None
    for c in range(_BPW // _CH):
        d = pltpu.async_copy(lab_hbm.at[idx_v.at[pl.ds(c * _CH, _CH)]],
                             rows2_v.at[c % 2], sems[c % 2])
        if prev is not None:
            pc, pd = prev
            pd.wait()
            pltpu.sync_copy(rows2_v.at[pc % 2],
                            out2_hbm.at[pl.ds(base + pc * _CH, _CH)])
        prev = (c, d)
    pc, pd = prev
    pd.wait()
    pltpu.sync_copy(rows2_v.at[pc % 2], out2_hbm.at[pl.ds(base + pc * _CH, _CH)])

    for d in in2_descs:
        d.wait()
    pltpu.sync_copy(rows1_v, out1_hbm.at[pl.ds(base, _BPW)])


@functools.cache
def _sel_gather():
    return functools.partial(
        pl.kernel,
        mesh=plsc.VectorSubcoreMesh(core_axis_name="c", subcore_axis_name="s"),
        out_type=(
            jax.ShapeDtypeStruct((_K, _D), jnp.float32),
            jax.ShapeDtypeStruct((_K, _CP), jnp.float32),
        ),
        scratch_types=[
            pltpu.VMEM((_BPW,), jnp.int32),
            pltpu.VMEM((_BPW, _D), jnp.float32),
            pltpu.VMEM((2, _CH, _CP), jnp.float32),
            pltpu.SemaphoreType.DMA,
            pltpu.SemaphoreType.DMA,
            pltpu.SemaphoreType.DMA,
        ],
        compiler_params=pltpu.CompilerParams(needs_layout_passes=False),
    )(_gather_body)


def kernel(input1, input2, score1, score2, label):
    del input1
    loss, labrm = _loss_fn(score1, score2, label)
    key = lax.bitcast_convert_type(loss, jnp.int32)
    idx = _radix_sort()(key)
    inputss, labelss_p = _sel_gather()(idx, input2, labrm)
    return (inputss, labelss_p[:, :_C])


# speedup vs baseline: 1.1534x; 1.0233x over previous
import functools

import jax
import jax.numpy as jnp
from jax import lax
from jax.experimental import pallas as pl
from jax.experimental.pallas import tpu as pltpu
from jax.experimental.pallas import tpu_sc as plsc

_N = 16384
_C = 1000
_D = 128
_K = _N // 2


_BN = 1024
_CP = 1024


def _loss_body(s1_ref, s2_ref, lab_ref, loss_ref, labrm_ref):
    lab = lab_ref[...]
    a = jnp.sum(lab * jnp.log(s2_ref[...]), axis=0)
    b = jnp.sum(lab * jnp.log(s1_ref[...]), axis=0)
    loss_ref[...] = jnp.abs(-a + b)
    labrm_ref[...] = jnp.concatenate(
        [lab.T, jnp.zeros((_BN, _CP - _C), jnp.float32)], axis=1)


def _loss_fn(score1, score2, label):
    return pl.pallas_call(
        _loss_body,
        grid=(_N // _BN,),
        in_specs=[pl.BlockSpec((_C, _BN), lambda i: (0, i))] * 3,
        out_specs=[
            pl.BlockSpec((_BN,), lambda i: (i,)),
            pl.BlockSpec((_BN, _CP), lambda i: (i, 0)),
        ],
        out_shape=[
            jax.ShapeDtypeStruct((_N,), jnp.float32),
            jax.ShapeDtypeStruct((_N, _CP), jnp.float32),
        ],
    )(score1.T, score2.T, label.T)


_NT = 16
_EPT = _N // _NT
_NB = 2048
_BAT = 64


def _radix_body(key_hbm, idx_hbm, cur_k, cur_v, counters, allhist,
                posb, kb, vb, obuf, hists_sh, s1k, s1v, s2k, s2v, sem):
    cid = lax.axis_index("c")
    sid = lax.axis_index("s")

    @pl.when(cid == 0)
    def _radix():
        base = sid * _EPT
        zero16 = jnp.zeros((16,), jnp.int32)
        iota16 = lax.iota(jnp.int32, 16)
        calr, _ = plsc.scan_count(zero16)
        adj = calr[0]

        pltpu.sync_copy(key_hbm.at[pl.ds(base, _EPT)], cur_k)

        def vinit(t, carry):
            cur_v[pl.ds(t * 16, 16)] = base + t * 16 + iota16
            return carry

        lax.fori_loop(0, _EPT // 16, vinit, 0)

        def run_pass(digit_fn, dst_k, dst_v, last):
            def zstep(t, carry):
                counters[pl.ds(t * 16, 16)] = zero16
                return carry

            lax.fori_loop(0, _NB // 16, zstep, 0)

            def hstep(t, carry):
                d = digit_fn(cur_k[pl.ds(t * 16, 16)])
                dupr, islast = plsc.scan_count(d)
                cnt = plsc.load_gather(counters, [d])
                plsc.store_scatter(counters, [d], cnt + (dupr - adj) + 1,
                                   mask=islast)
                return carry

            lax.fori_loop(0, _EPT // 16, hstep, 0)

            pltpu.sync_copy(counters, hists_sh.at[sid])
            plsc.subcore_barrier()
            pltpu.sync_copy(hists_sh, allhist)

            def ostep(c, carry):
                sl = pl.ds(c * 16, 16)
                tot = zero16
                below = zero16
                for t in range(_NT):
                    row = allhist[t, sl]
                    tot = tot + row
                    below = below + jnp.where(t < sid, row, 0)
                incl = plsc.cumsum(tot)
                counters[sl] = (incl - tot) + below + carry
                return carry + jnp.sum(tot)

            lax.fori_loop(0, _NB // 16, ostep, 0)

            ndma = 1 if last else 2

            def fill_and_fire(b, half):
                for m in range(_BAT // 16):
                    sl = pl.ds(b * _BAT + m * 16, 16)
                    osl = pl.ds(m * 16, 16)
                    ke = cur_k[sl]
                    ve = cur_v[sl]
                    d = digit_fn(ke)
                    dupr, islast = plsc.scan_count(d)
                    cnt = plsc.load_gather(counters, [d])
                    pos = cnt + (dupr - adj)
                    plsc.store_scatter(counters, [d], pos + 1, mask=islast)
                    posb[half, osl] = pos
                    if not last:
                        kb[half, osl] = ke
                    vb[half, osl] = ve
                if not last:
                    pltpu.async_copy(kb.at[half], dst_k.at[posb.at[half]], sem)
                pltpu.async_copy(vb.at[half], dst_v.at[posb.at[half]], sem)

            def drain_one():
                for _ in range(ndma):
                    pltpu.make_async_copy(
                        key_hbm.at[pl.ds(0, _BAT)], vb.at[0], sem).wait()

            def bstep(j, carry):
                for half in range(2):
                    @pl.when(j > 0)
                    def _():
                        drain_one()

                    fill_and_fire(j * 2 + half, half)
                return carry

            lax.fori_loop(0, _EPT // _BAT // 2, bstep, 0)
            drain_one()
            drain_one()
            plsc.subcore_barrier()

        run_pass(lambda k: k & (_NB - 1), s1k, s1v, False)
        pltpu.sync_copy(s1k.at[pl.ds(base, _EPT)], cur_k)
        pltpu.sync_copy(s1v.at[pl.ds(base, _EPT)], cur_v)
        plsc.subcore_barrier()
        run_pass(lambda k: (k >> 11) & (_NB - 1), s2k, s2v, False)
        pltpu.sync_copy(s2k.at[pl.ds(base, _EPT)], cur_k)
        pltpu.sync_copy(s2v.at[pl.ds(base, _EPT)], cur_v)
        plsc.subcore_barrier()
        run_pass(lambda k: k >> 22, s1k, s1v, True)

        pltpu.sync_copy(s1v.at[pl.ds(sid * (_K // _NT), _K // _NT)], obuf)
        pltpu.sync_copy(obuf, idx_hbm.at[pl.ds(sid * (_K // _NT), _K // _NT)])


@functools.cache
def _radix_sort():
    return functools.partial(
        pl.kernel,
        mesh=plsc.VectorSubcoreMesh(core_axis_name="c", subcore_axis_name="s"),
        out_type=jax.ShapeDtypeStruct((_K,), jnp.int32),
        scratch_types=[
            pltpu.VMEM((_EPT,), jnp.int32),
            pltpu.VMEM((_EPT,), jnp.int32),
            pltpu.VMEM((_NB,), jnp.int32),
            pltpu.VMEM((_NT, _NB), jnp.int32),
            pltpu.VMEM((2, _BAT), jnp.int32),
            pltpu.VMEM((2, _BAT), jnp.int32),
            pltpu.VMEM((2, _BAT), jnp.int32),
            pltpu.VMEM((_K // _NT,), jnp.int32),
            pltpu.VMEM_SHARED((_NT, _NB), jnp.int32),
            pltpu.VMEM_SHARED((_N,), jnp.int32),
            pltpu.VMEM_SHARED((_N,), jnp.int32),
            pltpu.VMEM_SHARED((_N,), jnp.int32),
            pltpu.VMEM_SHARED((_N,), jnp.int32),
            pltpu.SemaphoreType.DMA,
        ],
        compiler_params=pltpu.CompilerParams(needs_layout_passes=False),
    )(_radix_body)


_NW = 32
_BPW = _K // _NW
_CH = 32
_CH1 = 128


def _gather_body(idx_hbm, in2_hbm, lab_hbm, out1_hbm, out2_hbm,
                 idx_v, rows1_v, rows2_v, sem1, sem2a, sem2b):
    wid = lax.axis_index("s") * 2 + lax.axis_index("c")
    base = wid * _BPW
    pltpu.sync_copy(idx_hbm.at[pl.ds(base, _BPW)], idx_v)

    in2_descs = [
        pltpu.async_copy(in2_hbm.at[idx_v.at[pl.ds(c * _CH1, _CH1)]],
                         rows1_v.at[pl.ds(c * _CH1, _CH1)], sem1)
        for c in range(_BPW // _CH1)
    ]

    sems = (sem2a, sem2b)
    prev = None
    for c in range(_BPW // _CH):
        d = pltpu.async_copy(lab_hbm.at[idx_v.at[pl.ds(c * _CH, _CH)]],
                             rows2_v.at[c % 2], sems[c % 2])
        if prev is not None:
            pc, pd = prev
            pd.wait()
            pltpu.sync_copy(rows2_v.at[pc % 2],
                            out2_hbm.at[pl.ds(base + pc * _CH, _CH)])
        prev = (c, d)
    pc, pd = prev
    pd.wait()
    pltpu.sync_copy(rows2_v.at[pc % 2], out2_hbm.at[pl.ds(base + pc * _CH, _CH)])

    for d in in2_descs:
        d.wait()
    pltpu.sync_copy(rows1_v, out1_hbm.at[pl.ds(base, _BPW)])


@functools.cache
def _sel_gather():
    return functools.partial(
        pl.kernel,
        mesh=plsc.VectorSubcoreMesh(core_axis_name="c", subcore_axis_name="s"),
        out_type=(
            jax.ShapeDtypeStruct((_K, _D), jnp.float32),
            jax.ShapeDtypeStruct((_K, _CP), jnp.float32),
        ),
        scratch_types=[
            pltpu.VMEM((_BPW,), jnp.int32),
            pltpu.VMEM((_BPW, _D), jnp.float32),
            pltpu.VMEM((2, _CH, _CP), jnp.float32),
            pltpu.SemaphoreType.DMA,
            pltpu.SemaphoreType.DMA,
            pltpu.SemaphoreType.DMA,
        ],
        compiler_params=pltpu.CompilerParams(needs_layout_passes=False),
    )(_gather_body)


def kernel(input1, input2, score1, score2, label):
    del input1
    loss, labrm = _loss_fn(score1, score2, label)
    key = lax.bitcast_convert_type(loss, jnp.int32)
    idx = _radix_sort()(key)
    inputss, labelss_p = _sel_gather()(idx, input2, labrm)
    return (inputss, labelss_p[:, :_C])
